# baseline (speedup 1.0000x reference)
import functools

import jax
import jax.numpy as jnp
from jax.experimental import pallas as pl
from jax.experimental.pallas import tpu as pltpu

B, N, D = 4, 2048, 128
UNITS = 128


def _body(x_ref, adj_hbm, wm_ref, wu_ref, out_ref, msgt_ref, abuf, sem):
    b = pl.program_id(0)
    adj_f8 = adj_hbm.bitcast(jnp.float8_e4m3fn)

    def _start(step, slot):
        pltpu.make_async_copy(adj_f8.at[step], abuf.at[slot], sem.at[slot]
                              ).start()

    @pl.when(b == 0)
    def _prologue():
        _start(0, 0)

    @pl.when(b + 1 < B)
    def _prefetch():
        _start(b + 1, (b + 1) % 2)

    msg = jnp.dot(
        x_ref[0], wm_ref[...], preferred_element_type=jnp.float32
    )
    msgt_ref[0:UNITS, :] = msg.T.astype(jnp.float8_e4m3fn)
    msgt_ref[UNITS : UNITS + 8, :] = jnp.ones((8, N), jnp.float8_e4m3fn)

    pltpu.make_async_copy(adj_f8.at[b], abuf.at[b % 2], sem.at[b % 2]).wait()
    a = abuf[b % 2]
    res = jax.lax.dot_general(
        msgt_ref[...], a, (((1,), (0,)), ((), ())),
        preferred_element_type=jnp.float32,
    )
    agg = res[0:UNITS, :]
    deg = res[UNITS : UNITS + 1, :]
    inv = jnp.where(deg > 0.0, 1.0 / jnp.maximum(deg, 2.0**-9), 0.0)
    mean_t = agg * inv
    upd = jnp.dot(
        x_ref[0], wu_ref[...], preferred_element_type=jnp.float32
    )
    out_ref[0] = upd + mean_t.T


@jax.jit
def kernel(x, adj, W_msg, W_upd):
    grid = (B,)
    return pl.pallas_call(
        _body,
        grid=grid,
        in_specs=[
            pl.BlockSpec((1, N, D), lambda b: (b, 0, 0)),
            pl.BlockSpec(memory_space=pltpu.MemorySpace.HBM),
            pl.BlockSpec((D, UNITS), lambda b: (0, 0)),
            pl.BlockSpec((D, UNITS), lambda b: (0, 0)),
        ],
        out_specs=pl.BlockSpec((1, N, UNITS), lambda b: (b, 0, 0)),
        out_shape=jax.ShapeDtypeStruct((B, N, UNITS), jnp.float32),
        scratch_shapes=[
            pltpu.VMEM((UNITS + 8, N), jnp.float8_e4m3fn),
            pltpu.VMEM((2, N, N), jnp.float8_e4m3fn),
            pltpu.SemaphoreType.DMA((2,)),
        ],
    )(x, adj.view(jnp.int8), W_msg, W_upd)
